# Initial kernel scaffold; baseline (speedup 1.0000x reference)
import functools

import jax
import jax.numpy as jnp
from jax.experimental import pallas as pl

E = 8
K = 2
D = 1024
H = 2048
O = 1024
N = 2048

BM = 512


def _moe_block(x_ref, wg_ref, bg_ref, w1_ref, b1_ref, w2_ref, b2_ref, out_ref):
    e = pl.program_id(1)
    xb = x_ref[...]

    logits = jax.lax.dot_general(
        xb, wg_ref[...], (((1,), (1,)), ((), ())),
        preferred_element_type=jnp.float32) + bg_ref[...]
    colid = jax.lax.broadcasted_iota(jnp.int32, (BM, E), 1)
    v1 = jnp.max(logits, axis=1, keepdims=True)
    i1 = jnp.min(jnp.where(logits == v1, colid, E), axis=1, keepdims=True)
    masked = jnp.where(colid == i1, -jnp.inf, logits)
    v2 = jnp.max(masked, axis=1, keepdims=True)
    i2 = jnp.min(jnp.where(masked == v2, colid, E), axis=1, keepdims=True)
    t = jnp.exp(v2 - v1)
    w_top1 = 1.0 / (1.0 + t)
    w_top2 = 1.0 - w_top1
    c_e = jnp.where(i1 == e, w_top1, jnp.where(i2 == e, w_top2, 0.0))

    h = jax.lax.dot_general(
        xb, w1_ref[0], (((1,), (1,)), ((), ())),
        preferred_element_type=jnp.float32) + b1_ref[...]
    h = jnp.maximum(h, 0.0)
    y = jax.lax.dot_general(
        h, w2_ref[0], (((1,), (1,)), ((), ())),
        preferred_element_type=jnp.float32) + b2_ref[...]
    val = c_e * y

    @pl.when(e == 0)
    def _init():
        out_ref[...] = val

    @pl.when(e != 0)
    def _acc():
        out_ref[...] += val


@jax.jit
def kernel(x, Wg, bg, W1, b1, W2, b2):
    grid = (N // BM, E)
    out = pl.pallas_call(
        _moe_block,
        grid=grid,
        in_specs=[
            pl.BlockSpec((BM, D), lambda i, e: (i, 0)),
            pl.BlockSpec((E, D), lambda i, e: (0, 0)),
            pl.BlockSpec((1, E), lambda i, e: (0, 0)),
            pl.BlockSpec((1, H, D), lambda i, e: (e, 0, 0)),
            pl.BlockSpec((1, H), lambda i, e: (e, 0)),
            pl.BlockSpec((1, O, H), lambda i, e: (e, 0, 0)),
            pl.BlockSpec((1, O), lambda i, e: (e, 0)),
        ],
        out_specs=pl.BlockSpec((BM, O), lambda i, e: (i, 0)),
        out_shape=jax.ShapeDtypeStruct((N, O), jnp.float32),
    )(x, Wg, bg.reshape(1, E), W1, b1, W2, b2)
    return out


# speedup vs baseline: 1.6949x; 1.6949x over previous
import functools

import jax
import jax.numpy as jnp
from jax.experimental import pallas as pl

E = 8
K = 2
D = 1024
H = 2048
O = 1024
N = 2048

BM = 512


def _moe_block(x_ref, wg_ref, bg_ref, w1_ref, b1_ref, w2_ref, b2_ref, out_ref):
    e = pl.program_id(1)
    xb = x_ref[...]

    logits = jax.lax.dot_general(
        xb, wg_ref[...], (((1,), (1,)), ((), ())),
        preferred_element_type=jnp.float32) + bg_ref[...]
    colid = jax.lax.broadcasted_iota(jnp.int32, (BM, E), 1)
    v1 = jnp.max(logits, axis=1, keepdims=True)
    i1 = jnp.min(jnp.where(logits == v1, colid, E), axis=1, keepdims=True)
    masked = jnp.where(colid == i1, -jnp.inf, logits)
    v2 = jnp.max(masked, axis=1, keepdims=True)
    i2 = jnp.min(jnp.where(masked == v2, colid, E), axis=1, keepdims=True)
    t = jnp.exp(v2 - v1)
    w_top1 = 1.0 / (1.0 + t)
    w_top2 = 1.0 - w_top1
    c_e = jnp.where(i1 == e, w_top1, jnp.where(i2 == e, w_top2, 0.0))

    h = jax.lax.dot_general(
        xb, w1_ref[0], (((1,), (1,)), ((), ())),
        preferred_element_type=jnp.float32) + b1_ref[0]
    h = jnp.maximum(h, 0.0)
    y = jax.lax.dot_general(
        h, w2_ref[0], (((1,), (1,)), ((), ())),
        preferred_element_type=jnp.float32) + b2_ref[0]
    val = c_e * y

    @pl.when(e == 0)
    def _init():
        out_ref[...] = val

    @pl.when(e != 0)
    def _acc():
        out_ref[...] += val


@jax.jit
def kernel(x, Wg, bg, W1, b1, W2, b2):
    grid = (N // BM, E)
    out = pl.pallas_call(
        _moe_block,
        grid=grid,
        in_specs=[
            pl.BlockSpec((BM, D), lambda i, e: (i, 0)),
            pl.BlockSpec((E, D), lambda i, e: (0, 0)),
            pl.BlockSpec((1, E), lambda i, e: (0, 0)),
            pl.BlockSpec((1, H, D), lambda i, e: (e, 0, 0)),
            pl.BlockSpec((1, 1, H), lambda i, e: (e, 0, 0)),
            pl.BlockSpec((1, O, H), lambda i, e: (e, 0, 0)),
            pl.BlockSpec((1, 1, O), lambda i, e: (e, 0, 0)),
        ],
        out_specs=pl.BlockSpec((BM, O), lambda i, e: (i, 0)),
        out_shape=jax.ShapeDtypeStruct((N, O), jnp.float32),
    )(x, Wg, bg.reshape(1, E), W1, b1.reshape(E, 1, H), W2, b2.reshape(E, 1, O))
    return out
